# baseline (speedup 1.0000x reference)
import jax
import jax.numpy as jnp
from jax import lax
from jax.experimental import pallas as pl
from jax.experimental.pallas import tpu as pltpu
from jax.experimental.pallas import tpu_sc as plsc

_IMG = 128
_NPIX = _IMG * _IMG
_K = 8
_SB = 128
_NC, _NS = 2, 16
_NW = _NC * _NS
_SPT = 512 // _NW


def _gather_sc(img_ref, gidx_ref, out_ref, rows_v, idxb, sem):
    cid = lax.axis_index("c")
    sid = lax.axis_index("s")
    wid = sid * _NC + cid
    pltpu.sync_copy(gidx_ref.at[pl.ds(wid * 3 * _SPT, 3 * _SPT)], idxb)
    pltpu.async_copy(img_ref.at[idxb], rows_v, sem).wait()
    pltpu.sync_copy(rows_v, out_ref.at[wid])


def _topk_kernel(img_ref, rows_ref, ix_ref, tgt_ref,
                 d_ref, r1_ref, v1_ref, r2_ref, v2_ref, picks_ref):
    ix = ix_ref[0]
    lane = jax.lax.broadcasted_iota(jnp.int32, (_SB, _IMG), 1)
    sel = lane == ix
    zed = jnp.zeros((_SB, _IMG), jnp.float32)
    c0 = jnp.sum(jnp.where(sel, rows_ref[0, 0], zed), axis=1, keepdims=True)
    c1 = jnp.sum(jnp.where(sel, rows_ref[0, 1], zed), axis=1, keepdims=True)
    c2 = jnp.sum(jnp.where(sel, rows_ref[0, 2], zed), axis=1, keepdims=True)
    inf = jnp.float32(jnp.inf)
    r1 = jnp.full((_SB, _IMG), inf, jnp.float32)
    v1 = jnp.zeros((_SB, _IMG), jnp.int32)
    r2 = jnp.full((_SB, _IMG), inf, jnp.float32)
    v2 = jnp.zeros((_SB, _IMG), jnp.int32)
    for v in range(_IMG):
        sl = slice(v * _IMG, (v + 1) * _IMG)
        x = (jnp.abs(img_ref[0, 0, sl][None, :] - c0) +
             jnp.abs(img_ref[0, 1, sl][None, :] - c1) +
             jnp.abs(img_ref[0, 2, sl][None, :] - c2)) / 3.0
        d_ref[v] = x
        vc = jnp.full((_SB, _IMG), v, jnp.int32)
        c = x < r1
        dem = jnp.where(c, r1, x)
        demi = jnp.where(c, v1, vc)
        r1 = jnp.where(c, x, r1)
        v1 = jnp.where(c, vc, v1)
        cc = dem < r2
        r2 = jnp.where(cc, dem, r2)
        v2 = jnp.where(cc, demi, v2)
    r1_ref[...] = r1
    v1_ref[...] = v1
    r2_ref[...] = r2
    v2_ref[...] = v2
    picks = []
    for k in range(_K):
        if k >= 2:
            trig = jnp.any(r1_ref[...] == inf)

            @pl.when(trig)
            def _rebuild(k=k):
                def body(v, carry):
                    br1, bv1, br2, bv2 = carry
                    x = d_ref[v]
                    pv = v * _IMG + lane
                    ex = pv == picks_ref[:, 0:1]
                    for j in range(1, k):
                        ex = ex | (pv == picks_ref[:, j:j + 1])
                    x = jnp.where(ex, inf, x)
                    c = x < br1
                    dem = jnp.where(c, br1, x)
                    demi = jnp.where(c, bv1, v)
                    br1 = jnp.where(c, x, br1)
                    bv1 = jnp.where(c, v, bv1)
                    cc = dem < br2
                    br2 = jnp.where(cc, dem, br2)
                    bv2 = jnp.where(cc, demi, bv2)
                    return br1, bv1, br2, bv2

                init = (jnp.full((_SB, _IMG), inf, jnp.float32),
                        jnp.zeros((_SB, _IMG), jnp.int32),
                        jnp.full((_SB, _IMG), inf, jnp.float32),
                        jnp.zeros((_SB, _IMG), jnp.int32))
                fr1, fv1, fr2, fv2 = jax.lax.fori_loop(0, _IMG, body, init)
                r1_ref[...] = fr1
                v1_ref[...] = fv1
                r2_ref[...] = fr2
                v2_ref[...] = fv2

        r1 = r1_ref[...]
        v1 = v1_ref[...]
        m = jnp.min(r1, axis=1, keepdims=True)
        cand = jnp.where(r1 == m, v1 * _IMG + lane, jnp.int32(2 ** 30))
        p = jnp.min(cand, axis=1)
        picks.append(p)
        picks_ref[:, k:k + 1] = p[:, None]
        jm = lane == (p % _IMG)[:, None]
        r2v = r2_ref[...]
        v2v = v2_ref[...]
        r1_ref[...] = jnp.where(jm, r2v, r1)
        v1_ref[...] = jnp.where(jm, v2v, v1)
        r2_ref[...] = jnp.where(jm, inf, r2v)
    xs = [(p % _IMG).astype(jnp.float32) / _IMG for p in picks]
    ys = [(p // _IMG).astype(jnp.float32) / _IMG for p in picks]
    tgt_ref[0] = jnp.stack(xs + ys, axis=1)


def _loss_kernel(tgt_ref, pxn_ref, pyn_ref, out_ref):
    tx = tgt_ref[:, 0, 0:_K]
    ty = tgt_ref[:, 0, _K:2 * _K]
    dx = pxn_ref[:, 0:1] - tx
    dy = pyn_ref[:, 0:1] - ty
    dist = jnp.sqrt(dx * dx + dy * dy)
    mn = jnp.min(dist, axis=1)
    out_ref[:, :] = (jnp.sum(mn) / jnp.float32(4 * (_IMG - 1))).reshape(1, 1)


def kernel(predictions, ref_imgs):
    bs, L, _ = predictions.shape
    pos = predictions[:, :, :2]
    tmp = pos.reshape(bs * L, 2)
    q = tmp.reshape(L, bs, 2).transpose(1, 0, 2)
    gx = 2.0 * q[:, :, 0] - 1.0
    gy = 2.0 * q[:, :, 1] - 1.0
    fx = ((gx + 1.0) * _IMG - 1.0) / 2.0
    fy = ((gy + 1.0) * _IMG - 1.0) / 2.0
    ix_all = jnp.clip(jnp.round(fx), 0, _IMG - 1).astype(jnp.int32)
    iy_all = jnp.clip(jnp.round(fy), 0, _IMG - 1).astype(jnp.int32)
    pix = (iy_all * _IMG + ix_all).reshape(bs * L)

    iy_flat = iy_all.reshape(bs * L)
    b_of = jnp.arange(bs * L) // L
    base = b_of * (3 * _IMG) + iy_flat
    per_tile = base.reshape(_NW, _SPT)
    gidx = (per_tile[:, None, :] +
            (jnp.arange(3) * _IMG)[None, :, None]).reshape(_NW * 3 * _SPT)
    sc_rows = pl.kernel(
        _gather_sc,
        out_type=jax.ShapeDtypeStruct((_NW, 3 * _SPT, _IMG), jnp.float32),
        mesh=plsc.VectorSubcoreMesh(core_axis_name="c", subcore_axis_name="s"),
        scratch_types=[
            pltpu.VMEM((3 * _SPT, _IMG), jnp.float32),
            pltpu.VMEM((3 * _SPT,), jnp.int32),
            pltpu.SemaphoreType.DMA,
        ],
    )(ref_imgs.reshape(bs * 3 * _IMG, _IMG), gidx.astype(jnp.int32))
    crows = sc_rows.reshape(_NW, 3, _SPT, _IMG).transpose(0, 2, 1, 3)
    crows = crows.reshape(bs, L, 3, _IMG).transpose(0, 2, 1, 3)
    ixr = ix_all.reshape(bs, L, 1)

    ngroups = L // _SB
    img_flat = ref_imgs.reshape(bs, 3, _NPIX)
    rspec = pl.BlockSpec((1, 3, _SB, _IMG), lambda b, g: (b, 0, g, 0))
    ispec = pl.BlockSpec((1, _SB, 1), lambda b, g: (b, g, 0))
    tgt = pl.pallas_call(
        _topk_kernel,
        grid=(bs, ngroups),
        in_specs=[
            pl.BlockSpec((1, 3, _NPIX), lambda b, g: (b, 0, 0)),
            rspec, ispec,
        ],
        out_specs=pl.BlockSpec((1, _SB, 2 * _K),
                               lambda b, g: (b * ngroups + g, 0, 0)),
        out_shape=jax.ShapeDtypeStruct((bs * L // _SB, _SB, 2 * _K),
                                       jnp.float32),
        compiler_params=pltpu.CompilerParams(
            dimension_semantics=("parallel", "parallel")),
        scratch_shapes=[
            pltpu.VMEM((_IMG, _SB, _IMG), jnp.float32),
            pltpu.VMEM((_SB, _IMG), jnp.float32),
            pltpu.VMEM((_SB, _IMG), jnp.int32),
            pltpu.VMEM((_SB, _IMG), jnp.float32),
            pltpu.VMEM((_SB, _IMG), jnp.int32),
            pltpu.VMEM((_SB, _K), jnp.int32),
        ],
    )(img_flat, crows, ixr)

    n1 = bs * (L - 1)
    tgt_prev = tgt.reshape(bs, L, 2 * _K)[:, :L - 1].reshape(n1, 1, 2 * _K)
    pxn = predictions[:, 1:, 0].reshape(n1, 1)
    pyn = predictions[:, 1:, 1].reshape(n1, 1)
    res = pl.pallas_call(
        _loss_kernel,
        out_shape=jax.ShapeDtypeStruct((1, 1), jnp.float32),
    )(tgt_prev, pxn, pyn)
    return res[0, 0]


# speedup vs baseline: 3.6158x; 3.6158x over previous
import jax
import jax.numpy as jnp
from jax import lax
from jax.experimental import pallas as pl
from jax.experimental.pallas import tpu as pltpu
from jax.experimental.pallas import tpu_sc as plsc

_IMG = 128
_NPIX = _IMG * _IMG
_K = 8
_SB = 128
_NC, _NS = 2, 16
_NW = _NC * _NS
_SPT = 512 // _NW


def _gather_sc(img_ref, gidx_ref, out_ref, rows_v, idxb, sem):
    cid = lax.axis_index("c")
    sid = lax.axis_index("s")
    wid = sid * _NC + cid
    pltpu.sync_copy(gidx_ref.at[pl.ds(wid * 3 * _SPT, 3 * _SPT)], idxb)
    pltpu.async_copy(img_ref.at[idxb], rows_v, sem).wait()
    pltpu.sync_copy(rows_v, out_ref.at[wid])


def _topk_kernel(img_ref, rows_ref, ix_ref, tgt_ref,
                 d_ref, r1_ref, v1_ref, r2_ref, v2_ref, r3_ref, v3_ref,
                 picks_ref):
    ix = ix_ref[0]
    lane = jax.lax.broadcasted_iota(jnp.int32, (_SB, _IMG), 1)
    sel = lane == ix
    zed = jnp.zeros((_SB, _IMG), jnp.float32)
    c0 = jnp.sum(jnp.where(sel, rows_ref[0, 0], zed), axis=1, keepdims=True)
    c1 = jnp.sum(jnp.where(sel, rows_ref[0, 1], zed), axis=1, keepdims=True)
    c2 = jnp.sum(jnp.where(sel, rows_ref[0, 2], zed), axis=1, keepdims=True)
    inf = jnp.float32(jnp.inf)
    r1 = jnp.full((_SB, _IMG), inf, jnp.float32)
    v1 = jnp.zeros((_SB, _IMG), jnp.int32)
    r2 = jnp.full((_SB, _IMG), inf, jnp.float32)
    v2 = jnp.zeros((_SB, _IMG), jnp.int32)
    r3 = jnp.full((_SB, _IMG), inf, jnp.float32)
    v3 = jnp.zeros((_SB, _IMG), jnp.int32)
    for v in range(_IMG):
        sl = slice(v * _IMG, (v + 1) * _IMG)
        x = (jnp.abs(img_ref[0, 0, sl][None, :] - c0) +
             jnp.abs(img_ref[0, 1, sl][None, :] - c1) +
             jnp.abs(img_ref[0, 2, sl][None, :] - c2)) / 3.0
        d_ref[v] = x
        vc = jnp.full((_SB, _IMG), v, jnp.int32)
        c = x < r1
        dem = jnp.where(c, r1, x)
        demi = jnp.where(c, v1, vc)
        r1 = jnp.where(c, x, r1)
        v1 = jnp.where(c, vc, v1)
        cc = dem < r2
        dem2 = jnp.where(cc, r2, dem)
        dem2i = jnp.where(cc, v2, demi)
        r2 = jnp.where(cc, dem, r2)
        v2 = jnp.where(cc, demi, v2)
        ccc = dem2 < r3
        r3 = jnp.where(ccc, dem2, r3)
        v3 = jnp.where(ccc, dem2i, v3)
    r1_ref[...] = r1
    v1_ref[...] = v1
    r2_ref[...] = r2
    v2_ref[...] = v2
    r3_ref[...] = r3
    v3_ref[...] = v3
    picks = []
    for k in range(_K):
        if k >= 3:
            trig = jnp.any(r1_ref[...] == inf)

            @pl.when(trig)
            def _rebuild(k=k):
                def body(v, carry):
                    br1, bv1, br2, bv2 = carry
                    x = d_ref[v]
                    pv = v * _IMG + lane
                    ex = pv == picks_ref[:, 0:1]
                    for j in range(1, k):
                        ex = ex | (pv == picks_ref[:, j:j + 1])
                    x = jnp.where(ex, inf, x)
                    c = x < br1
                    dem = jnp.where(c, br1, x)
                    demi = jnp.where(c, bv1, v)
                    br1 = jnp.where(c, x, br1)
                    bv1 = jnp.where(c, v, bv1)
                    cc = dem < br2
                    br2 = jnp.where(cc, dem, br2)
                    bv2 = jnp.where(cc, demi, bv2)
                    return br1, bv1, br2, bv2

                init = (jnp.full((_SB, _IMG), inf, jnp.float32),
                        jnp.zeros((_SB, _IMG), jnp.int32),
                        jnp.full((_SB, _IMG), inf, jnp.float32),
                        jnp.zeros((_SB, _IMG), jnp.int32))
                fr1, fv1, fr2, fv2 = jax.lax.fori_loop(0, _IMG, body, init)
                r1_ref[...] = fr1
                v1_ref[...] = fv1
                r2_ref[...] = fr2
                v2_ref[...] = fv2
                r3_ref[...] = jnp.full((_SB, _IMG), inf, jnp.float32)
                v3_ref[...] = jnp.zeros((_SB, _IMG), jnp.int32)

        r1 = r1_ref[...]
        v1 = v1_ref[...]
        m = jnp.min(r1, axis=1, keepdims=True)
        cand = jnp.where(r1 == m, v1 * _IMG + lane, jnp.int32(2 ** 30))
        p = jnp.min(cand, axis=1)
        picks.append(p)
        picks_ref[:, k:k + 1] = p[:, None]
        jm = lane == (p % _IMG)[:, None]
        r2v = r2_ref[...]
        v2v = v2_ref[...]
        r3v = r3_ref[...]
        v3v = v3_ref[...]
        r1_ref[...] = jnp.where(jm, r2v, r1)
        v1_ref[...] = jnp.where(jm, v2v, v1)
        r2_ref[...] = jnp.where(jm, r3v, r2v)
        v2_ref[...] = jnp.where(jm, v3v, v2v)
        r3_ref[...] = jnp.where(jm, inf, r3v)
    xs = [(p % _IMG).astype(jnp.float32) / _IMG for p in picks]
    ys = [(p // _IMG).astype(jnp.float32) / _IMG for p in picks]
    tgt_ref[0] = jnp.stack(xs + ys, axis=1)


def _loss_kernel(tgt_ref, pxn_ref, pyn_ref, out_ref):
    tx = tgt_ref[:, 0, 0:_K]
    ty = tgt_ref[:, 0, _K:2 * _K]
    dx = pxn_ref[:, 0:1] - tx
    dy = pyn_ref[:, 0:1] - ty
    dist = jnp.sqrt(dx * dx + dy * dy)
    mn = jnp.min(dist, axis=1)
    out_ref[:, :] = (jnp.sum(mn) / jnp.float32(4 * (_IMG - 1))).reshape(1, 1)


def kernel(predictions, ref_imgs):
    bs, L, _ = predictions.shape
    pos = predictions[:, :, :2]
    tmp = pos.reshape(bs * L, 2)
    q = tmp.reshape(L, bs, 2).transpose(1, 0, 2)
    gx = 2.0 * q[:, :, 0] - 1.0
    gy = 2.0 * q[:, :, 1] - 1.0
    fx = ((gx + 1.0) * _IMG - 1.0) / 2.0
    fy = ((gy + 1.0) * _IMG - 1.0) / 2.0
    ix_all = jnp.clip(jnp.round(fx), 0, _IMG - 1).astype(jnp.int32)
    iy_all = jnp.clip(jnp.round(fy), 0, _IMG - 1).astype(jnp.int32)
    pix = (iy_all * _IMG + ix_all).reshape(bs * L)

    iy_flat = iy_all.reshape(bs * L)
    b_of = jnp.arange(bs * L) // L
    base = b_of * (3 * _IMG) + iy_flat
    per_tile = base.reshape(_NW, _SPT)
    gidx = (per_tile[:, None, :] +
            (jnp.arange(3) * _IMG)[None, :, None]).reshape(_NW * 3 * _SPT)
    sc_rows = pl.kernel(
        _gather_sc,
        out_type=jax.ShapeDtypeStruct((_NW, 3 * _SPT, _IMG), jnp.float32),
        mesh=plsc.VectorSubcoreMesh(core_axis_name="c", subcore_axis_name="s"),
        scratch_types=[
            pltpu.VMEM((3 * _SPT, _IMG), jnp.float32),
            pltpu.VMEM((3 * _SPT,), jnp.int32),
            pltpu.SemaphoreType.DMA,
        ],
    )(ref_imgs.reshape(bs * 3 * _IMG, _IMG), gidx.astype(jnp.int32))
    crows = sc_rows.reshape(_NW, 3, _SPT, _IMG).transpose(0, 2, 1, 3)
    crows = crows.reshape(bs, L, 3, _IMG).transpose(0, 2, 1, 3)
    ixr = ix_all.reshape(bs, L, 1)

    ngroups = L // _SB
    img_flat = ref_imgs.reshape(bs, 3, _NPIX)
    rspec = pl.BlockSpec((1, 3, _SB, _IMG), lambda b, g: (b, 0, g, 0))
    ispec = pl.BlockSpec((1, _SB, 1), lambda b, g: (b, g, 0))
    tgt = pl.pallas_call(
        _topk_kernel,
        grid=(bs, ngroups),
        in_specs=[
            pl.BlockSpec((1, 3, _NPIX), lambda b, g: (b, 0, 0)),
            rspec, ispec,
        ],
        out_specs=pl.BlockSpec((1, _SB, 2 * _K),
                               lambda b, g: (b * ngroups + g, 0, 0)),
        out_shape=jax.ShapeDtypeStruct((bs * L // _SB, _SB, 2 * _K),
                                       jnp.float32),
        compiler_params=pltpu.CompilerParams(
            dimension_semantics=("parallel", "parallel")),
        scratch_shapes=[
            pltpu.VMEM((_IMG, _SB, _IMG), jnp.float32),
            pltpu.VMEM((_SB, _IMG), jnp.float32),
            pltpu.VMEM((_SB, _IMG), jnp.int32),
            pltpu.VMEM((_SB, _IMG), jnp.float32),
            pltpu.VMEM((_SB, _IMG), jnp.int32),
            pltpu.VMEM((_SB, _IMG), jnp.float32),
            pltpu.VMEM((_SB, _IMG), jnp.int32),
            pltpu.VMEM((_SB, _K), jnp.int32),
        ],
    )(img_flat, crows, ixr)

    n1 = bs * (L - 1)
    tgt_prev = tgt.reshape(bs, L, 2 * _K)[:, :L - 1].reshape(n1, 1, 2 * _K)
    pxn = predictions[:, 1:, 0].reshape(n1, 1)
    pyn = predictions[:, 1:, 1].reshape(n1, 1)
    res = pl.pallas_call(
        _loss_kernel,
        out_shape=jax.ShapeDtypeStruct((1, 1), jnp.float32),
    )(tgt_prev, pxn, pyn)
    return res[0, 0]
